# baseline (speedup 1.0000x reference)
import functools

import jax
import jax.numpy as jnp
from jax import lax
from jax.experimental import pallas as pl
from jax.experimental.pallas import tpu as pltpu
from jax.experimental.pallas import tpu_sc as plsc

_B = 16384
_D = 128
_LANES = 16
_NC = 2
_NS = 16
_NW = _NC * _NS
_BPW = _B // _NW
_CHUNK = 128
_NCHUNK = _BPW // _CHUNK
_GROUPS = _CHUNK // _LANES


def _sc_body(uids_ref, iids_ref, uemb_ref, iemb_ref, ubias_ref, ibias_ref,
             out_ref, uidx, iidx, u0, u1, i0, i1, ub, ib, scores,
             sem0, sem1, semb):
    w = lax.axis_index("s") * _NC + lax.axis_index("c")

    for c in range(_NCHUNK):
        pltpu.sync_copy(
            uids_ref.at[pl.ds(w * _BPW + c * _CHUNK, _CHUNK)], uidx.at[c])
        pltpu.sync_copy(
            iids_ref.at[pl.ds(w * _BPW + c * _CHUNK, _CHUNK)], iidx.at[c])

    bias_descs = []
    for c in range(_NCHUNK):
        sl = pl.ds(c * _CHUNK, _CHUNK)
        bias_descs.append(pltpu.async_copy(
            ubias_ref.at[uidx.at[pl.ds(c, 1)]], ub.at[:, sl], semb))
        bias_descs.append(pltpu.async_copy(
            ibias_ref.at[iidx.at[pl.ds(c, 1)]], ib.at[:, sl], semb))

    bufs = [(u0, i0, sem0), (u1, i1, sem1)]

    def issue(c):
        ubuf, ibuf, sem = bufs[c % 2]
        du = pltpu.async_copy(uemb_ref.at[uidx.at[c]], ubuf, sem)
        di = pltpu.async_copy(iemb_ref.at[iidx.at[c]], ibuf, sem)
        return du, di

    pend = [None] * _NCHUNK
    pend[0] = issue(0)
    for bd in bias_descs:
        bd.wait()

    lane = lax.iota(jnp.int32, _LANES)

    for c in range(_NCHUNK):
        du, di = pend[c]
        du.wait()
        di.wait()
        if c + 1 < _NCHUNK:
            pend[c + 1] = issue(c + 1)
        ubuf, ibuf, _ = bufs[c % 2]

        def group_body(g, _, ubuf=ubuf, ibuf=ibuf, c=c):
            def row_body(j, vec, ubuf=ubuf, ibuf=ibuf):
                r = g * _LANES + j
                p = ubuf[r, pl.ds(0, _LANES)] * ibuf[r, pl.ds(0, _LANES)]
                for k in range(1, _D // _LANES):
                    p = p + (ubuf[r, pl.ds(k * _LANES, _LANES)]
                             * ibuf[r, pl.ds(k * _LANES, _LANES)])
                t0 = (p[0] + p[1]) + (p[2] + p[3])
                t1 = (p[4] + p[5]) + (p[6] + p[7])
                t2 = (p[8] + p[9]) + (p[10] + p[11])
                t3 = (p[12] + p[13]) + (p[14] + p[15])
                s = (t0 + t1) + (t2 + t3)
                return jnp.where(lane == j, s, vec)

            vec = lax.fori_loop(0, _LANES, row_body,
                                jnp.zeros((_LANES,), jnp.float32))
            sl = pl.ds(g * _LANES, _LANES)
            bsl = pl.ds(c * _CHUNK + g * _LANES, _LANES)
            scores[c, sl] = vec + ub[0, bsl] + ib[0, bsl]
            return 0

        lax.fori_loop(0, _GROUPS, group_body, 0)

    for c in range(_NCHUNK):
        pltpu.sync_copy(
            scores.at[c], out_ref.at[pl.ds(w * _BPW + c * _CHUNK, _CHUNK)])


@functools.partial(
    pl.kernel,
    mesh=plsc.VectorSubcoreMesh(core_axis_name="c", subcore_axis_name="s"),
    out_type=jax.ShapeDtypeStruct((_B,), jnp.float32),
    scratch_types=[
        pltpu.VMEM((_NCHUNK, _CHUNK), jnp.int32),
        pltpu.VMEM((_NCHUNK, _CHUNK), jnp.int32),
        pltpu.VMEM((_CHUNK, _D), jnp.float32),
        pltpu.VMEM((_CHUNK, _D), jnp.float32),
        pltpu.VMEM((_CHUNK, _D), jnp.float32),
        pltpu.VMEM((_CHUNK, _D), jnp.float32),
        pltpu.VMEM((1, _BPW), jnp.float32),
        pltpu.VMEM((1, _BPW), jnp.float32),
        pltpu.VMEM((_NCHUNK, _CHUNK), jnp.float32),
        pltpu.SemaphoreType.DMA,
        pltpu.SemaphoreType.DMA,
        pltpu.SemaphoreType.DMA,
    ],
)
def _sc_scores(uids_ref, iids_ref, uemb_ref, iemb_ref, ubias_ref, ibias_ref,
               out_ref, *rest):
    _sc_body(uids_ref, iids_ref, uemb_ref, iemb_ref, ubias_ref, ibias_ref,
             out_ref, *rest)


def kernel(user_ids, item_ids, user_emb_table, item_emb_table,
           user_bias_table, item_bias_table):
    return _sc_scores(user_ids.astype(jnp.int32), item_ids.astype(jnp.int32),
                      user_emb_table, item_emb_table,
                      user_bias_table.reshape(1, -1),
                      item_bias_table.reshape(1, -1))


# speedup vs baseline: 1.0796x; 1.0796x over previous
import functools

import jax
import jax.numpy as jnp
from jax import lax
from jax.experimental import pallas as pl
from jax.experimental.pallas import tpu as pltpu
from jax.experimental.pallas import tpu_sc as plsc

_B = 16384
_D = 128
_LANES = 16
_NC = 2
_NS = 16
_NW = _NC * _NS
_BPW = _B // _NW
_CHUNK = 128
_NCHUNK = _BPW // _CHUNK
_GROUPS = _CHUNK // _LANES


def _sc_body(uids_ref, iids_ref, uemb_ref, iemb_ref, ubias_ref, ibias_ref,
             out_ref, uidx, iidx, u0, u1, i0, i1, ub, ib, scores,
             sem0, sem1, semb):
    w = lax.axis_index("s") * _NC + lax.axis_index("c")

    pltpu.sync_copy(uids_ref.at[pl.ds(w * _NCHUNK, _NCHUNK)], uidx)
    pltpu.sync_copy(iids_ref.at[pl.ds(w * _NCHUNK, _NCHUNK)], iidx)

    bias_descs = []
    for c in range(_NCHUNK):
        sl = pl.ds(c * _CHUNK, _CHUNK)
        bias_descs.append(pltpu.async_copy(
            ubias_ref.at[uidx.at[pl.ds(c, 1)]], ub.at[:, sl], semb))
        bias_descs.append(pltpu.async_copy(
            ibias_ref.at[iidx.at[pl.ds(c, 1)]], ib.at[:, sl], semb))

    bufs = [(u0, i0, sem0), (u1, i1, sem1)]

    def issue(c):
        ubuf, ibuf, sem = bufs[c % 2]
        du = pltpu.async_copy(uemb_ref.at[uidx.at[c]], ubuf, sem)
        di = pltpu.async_copy(iemb_ref.at[iidx.at[c]], ibuf, sem)
        return du, di

    pend = [None] * _NCHUNK
    pend[0] = issue(0)
    for bd in bias_descs:
        bd.wait()

    lane = lax.iota(jnp.int32, _LANES)

    for c in range(_NCHUNK):
        du, di = pend[c]
        du.wait()
        di.wait()
        if c + 1 < _NCHUNK:
            pend[c + 1] = issue(c + 1)
        ubuf, ibuf, _ = bufs[c % 2]

        def group_body(g, _, ubuf=ubuf, ibuf=ibuf, c=c):
            def row_body(j, vec, ubuf=ubuf, ibuf=ibuf):
                r = g * _LANES + j
                p = ubuf[r, pl.ds(0, _LANES)] * ibuf[r, pl.ds(0, _LANES)]
                for k in range(1, _D // _LANES):
                    p = p + (ubuf[r, pl.ds(k * _LANES, _LANES)]
                             * ibuf[r, pl.ds(k * _LANES, _LANES)])
                t0 = (p[0] + p[1]) + (p[2] + p[3])
                t1 = (p[4] + p[5]) + (p[6] + p[7])
                t2 = (p[8] + p[9]) + (p[10] + p[11])
                t3 = (p[12] + p[13]) + (p[14] + p[15])
                s = (t0 + t1) + (t2 + t3)
                return jnp.where(lane == j, s, vec)

            vec = lax.fori_loop(0, _LANES, row_body,
                                jnp.zeros((_LANES,), jnp.float32))
            sl = pl.ds(g * _LANES, _LANES)
            bsl = pl.ds(c * _CHUNK + g * _LANES, _LANES)
            scores[c, sl] = vec + ub[0, bsl] + ib[0, bsl]
            return 0

        lax.fori_loop(0, _GROUPS, group_body, 0)

    pltpu.sync_copy(scores, out_ref.at[pl.ds(w * _NCHUNK, _NCHUNK)])


@functools.partial(
    pl.kernel,
    mesh=plsc.VectorSubcoreMesh(core_axis_name="c", subcore_axis_name="s"),
    out_type=jax.ShapeDtypeStruct((_B // _CHUNK, _CHUNK), jnp.float32),
    scratch_types=[
        pltpu.VMEM((_NCHUNK, _CHUNK), jnp.int32),
        pltpu.VMEM((_NCHUNK, _CHUNK), jnp.int32),
        pltpu.VMEM((_CHUNK, _D), jnp.float32),
        pltpu.VMEM((_CHUNK, _D), jnp.float32),
        pltpu.VMEM((_CHUNK, _D), jnp.float32),
        pltpu.VMEM((_CHUNK, _D), jnp.float32),
        pltpu.VMEM((1, _BPW), jnp.float32),
        pltpu.VMEM((1, _BPW), jnp.float32),
        pltpu.VMEM((_NCHUNK, _CHUNK), jnp.float32),
        pltpu.SemaphoreType.DMA,
        pltpu.SemaphoreType.DMA,
        pltpu.SemaphoreType.DMA,
    ],
)
def _sc_scores(uids_ref, iids_ref, uemb_ref, iemb_ref, ubias_ref, ibias_ref,
               out_ref, *rest):
    _sc_body(uids_ref, iids_ref, uemb_ref, iemb_ref, ubias_ref, ibias_ref,
             out_ref, *rest)


def kernel(user_ids, item_ids, user_emb_table, item_emb_table,
           user_bias_table, item_bias_table):
    uids2 = user_ids.astype(jnp.int32).reshape(_B // _CHUNK, _CHUNK)
    iids2 = item_ids.astype(jnp.int32).reshape(_B // _CHUNK, _CHUNK)
    out2 = _sc_scores(uids2, iids2, user_emb_table, item_emb_table,
                      user_bias_table.reshape(1, -1),
                      item_bias_table.reshape(1, -1))
    return out2.reshape(_B)
